# speedup vs baseline: 1.9342x; 1.9342x over previous
import functools

import jax
import jax.numpy as jnp
from jax import lax
from jax.experimental import pallas as pl
from jax.experimental.pallas import tpu as pltpu

NBC = 160000
NAC = 320000
HC = 128
SBFC = 16
TW = 144
BN_S = 1.0 / (1.0 + 1e-3) ** 0.5

_F32 = jnp.float32


def _combine_body(w_im1, w2p_im, w_kj1, w2p_kj, wa_m1, wa_m2, wa_k1, wa_k2,
                  a_im, a_kj, wa_m, wa_k):
    s2 = jnp.float32(BN_S * BN_S)
    a_im[...] = s2 * jnp.dot(w_im1[...], w2p_im[...], preferred_element_type=_F32)
    a_kj[...] = s2 * jnp.dot(w_kj1[...], w2p_kj[...], preferred_element_type=_F32)
    wa_m[...] = jnp.dot(wa_m1[...], wa_m2[...], preferred_element_type=_F32)
    wa_k[...] = jnp.dot(wa_k1[...], wa_k2[...], preferred_element_type=_F32)


def _combine_weights(w_im1, w2p_im, w_kj1, w2p_kj, wa_m1, wa_m2, wa_k1, wa_k2):
    return pl.pallas_call(
        _combine_body,
        out_shape=[
            jax.ShapeDtypeStruct((2 * HC, TW), _F32),
            jax.ShapeDtypeStruct((2 * HC, TW), _F32),
            jax.ShapeDtypeStruct((SBFC, HC), _F32),
            jax.ShapeDtypeStruct((SBFC, HC), _F32),
        ],
    )(w_im1, w2p_im, w_kj1, w2p_kj, wa_m1, wa_m2, wa_k1, wa_k2)


_BM_T = 1600


def _tables_body(e, a_im, a_kj, tmi, tijm, tkj, tijk):
    eb = e[...]
    tmi[...] = jnp.dot(eb, a_im[:HC, :], preferred_element_type=_F32)
    tijm[...] = jnp.dot(eb, a_im[HC:, :], preferred_element_type=_F32)
    tkj[...] = jnp.dot(eb, a_kj[:HC, :], preferred_element_type=_F32)
    tijk[...] = jnp.dot(eb, a_kj[HC:, :], preferred_element_type=_F32)


def _make_tables(e, a_im, a_kj):
    grid = (NBC // _BM_T,)
    bs_out = pl.BlockSpec((_BM_T, TW), lambda i: (i, 0))
    return pl.pallas_call(
        _tables_body,
        grid=grid,
        in_specs=[
            pl.BlockSpec((_BM_T, HC), lambda i: (i, 0)),
            pl.BlockSpec((2 * HC, TW), lambda i: (0, 0)),
            pl.BlockSpec((2 * HC, TW), lambda i: (0, 0)),
        ],
        out_specs=[bs_out, bs_out, bs_out, bs_out],
        out_shape=[jax.ShapeDtypeStruct((NBC, TW), _F32) for _ in range(4)],
    )(e, a_im, a_kj)


_BM_A = 1600


def _aarr_body(sbf_m, sbf_k, wa_m, wa_k, am, ak):
    am[...] = jnp.dot(sbf_m[...], wa_m[...], preferred_element_type=_F32)
    ak[...] = jnp.dot(sbf_k[...], wa_k[...], preferred_element_type=_F32)


def _make_aarr(sbf_m, sbf_k, wa_m, wa_k):
    grid = (NAC // _BM_A,)
    return pl.pallas_call(
        _aarr_body,
        grid=grid,
        in_specs=[
            pl.BlockSpec((_BM_A, SBFC), lambda i: (i, 0)),
            pl.BlockSpec((_BM_A, SBFC), lambda i: (i, 0)),
            pl.BlockSpec((SBFC, HC), lambda i: (0, 0)),
            pl.BlockSpec((SBFC, HC), lambda i: (0, 0)),
        ],
        out_specs=[
            pl.BlockSpec((_BM_A, HC), lambda i: (i, 0)),
            pl.BlockSpec((_BM_A, HC), lambda i: (i, 0)),
        ],
        out_shape=[jax.ShapeDtypeStruct((NAC, HC), _F32) for _ in range(2)],
    )(sbf_m, sbf_k, wa_m, wa_k)


_BM_F = 1600


def _final_body(e, sm, sk, wpm, wpk, wr0a, br0a, wr0b, br0b,
                wr1a, br1a, wr1b, br1b, out):
    x = e[...] + jnp.dot(sm[...], wpm[...], preferred_element_type=_F32) \
        + jnp.dot(sk[...], wpk[...], preferred_element_type=_F32)
    y = jnp.dot(x, wr0a[...], preferred_element_type=_F32) + br0a[...]
    x = x + jnp.dot(y, wr0b[...], preferred_element_type=_F32) + br0b[...]
    y = jnp.dot(x, wr1a[...], preferred_element_type=_F32) + br1a[...]
    x = x + jnp.dot(y, wr1b[...], preferred_element_type=_F32) + br1b[...]
    out[...] = x


def _final(e, sm, sk, wpm, wpk, wr0a, br0a, wr0b, br0b, wr1a, br1a, wr1b, br1b):
    grid = (NBC // _BM_F,)
    bs_big = pl.BlockSpec((_BM_F, HC), lambda i: (i, 0))
    bs_w = pl.BlockSpec((HC, HC), lambda i: (0, 0))
    bs_b = pl.BlockSpec((1, HC), lambda i: (0, 0))
    return pl.pallas_call(
        _final_body,
        grid=grid,
        in_specs=[bs_big, bs_big, bs_big,
                  bs_w, bs_w, bs_w, bs_b, bs_w, bs_b, bs_w, bs_b, bs_w, bs_b],
        out_specs=bs_big,
        out_shape=jax.ShapeDtypeStruct((NBC, HC), _F32),
    )(e, sm, sk, wpm, wpk, wr0a, br0a, wr0b, br0b, wr1a, br1a, wr1b, br1b)


def _permute_pad_w2(w2):
    out = jnp.zeros((HC + 1, TW), _F32)
    out = out.at[:, :HC].set(w2[:, 1:])
    out = out.at[:, HC].set(w2[:, 0])
    return out


def kernel(bond_embedding, sbf_mij, sbf_kji, W_im1, W_im2, W_kj1, W_kj2,
           Wa_mij1, Wa_mij2, Wa_kji1, Wa_kji2, W_pre,
           Wr0a, br0a, Wr0b, br0b, Wr1a, br1a, Wr1b, br1b,
           bond_mi_id_for_angle_mij_list, bond_ij_id_for_angle_mij_list,
           bond_kj_id_for_angle_kji_list, bond_ij_id_for_angle_kji_list):
    e = bond_embedding
    mi = bond_mi_id_for_angle_mij_list
    ij_m = bond_ij_id_for_angle_mij_list
    kj = bond_kj_id_for_angle_kji_list
    ij_k = bond_ij_id_for_angle_kji_list

    w2p_im = _permute_pad_w2(W_im2)
    w2p_kj = _permute_pad_w2(W_kj2)
    wpm = BN_S * W_pre[:HC, :]
    wpk = BN_S * W_pre[HC:, :]
    b0a = br0a.reshape(1, HC)
    b0b = br0b.reshape(1, HC)
    b1a = br1a.reshape(1, HC)
    b1b = br1b.reshape(1, HC)

    a_im, a_kj, wa_m, wa_k = _combine_weights(
        W_im1, w2p_im, W_kj1, w2p_kj, Wa_mij1, Wa_mij2, Wa_kji1, Wa_kji2)
    tmi, tijm, tkj, tijk = _make_tables(e, a_im, a_kj)
    am, ak = _make_aarr(sbf_mij, sbf_kji, wa_m, wa_k)

    h_m = jnp.take(tmi, mi, axis=0) + jnp.take(tijm, ij_m, axis=0)
    msg_m = am * h_m[:, HC:HC + 1] * h_m[:, :HC]
    h_k = jnp.take(tkj, kj, axis=0) + jnp.take(tijk, ij_k, axis=0)
    msg_k = ak * h_k[:, HC:HC + 1] * h_k[:, :HC]
    sum_m = jax.ops.segment_sum(msg_m, ij_m, num_segments=NBC)
    sum_k = jax.ops.segment_sum(msg_k, ij_k, num_segments=NBC)

    return _final(e, sum_m, sum_k, wpm, wpk,
                  Wr0a, b0a, Wr0b, b0b, Wr1a, b1a, Wr1b, b1b)
